# baseline (speedup 1.0000x reference)
import functools

import jax
import jax.numpy as jnp
from jax import lax
from jax.experimental import pallas as pl
from jax.experimental.pallas import tpu as pltpu
from jax.experimental.pallas import tpu_sc as plsc

BB = 128
BBC = 1024
K = 8192
EMB = 32
H = 64


def _lrelu(v):
    return jnp.where(v > 0, v, 0.01 * v)


def _dot(a, b):
    return jnp.dot(a, b, preferred_element_type=jnp.float32)


CW = 128
NCH = K // CW


def _enc_kernel(x_ref, w1_ref, b1_ref, w2_ref, b2_ref, w3_ref, b3_ref,
                wemb_ref, wsq_ref, h_ref, idx_ref):
    x = x_ref[...]
    h1 = _lrelu(_dot(x, w1_ref[...]) + b1_ref[...])
    h2 = _lrelu(_dot(h1, w2_ref[...]) + b2_ref[...])
    h3 = _lrelu(_dot(h2, w3_ref[...]) + b3_ref[...])
    h_ref[...] = h3
    z0 = h3[:, :EMB]
    z1 = h3[:, EMB:]

    bb = h3.shape[0]
    lane = lax.broadcasted_iota(jnp.int32, (bb, CW), 1)

    def nearest_idx(z):
        zn = -2.0 * z
        acc = jnp.full((bb, CW), jnp.inf, jnp.float32)
        iacc = jnp.zeros((bb, CW), jnp.int32)
        for c in range(NCH):
            sl = slice(c * CW, (c + 1) * CW)
            d = _dot(zn, wemb_ref[:, sl]) + wsq_ref[:, sl]
            mask = d < acc
            acc = jnp.minimum(acc, d)
            iacc = jnp.where(mask, c, iacc)
        m = jnp.min(acc, axis=1, keepdims=True)
        j = iacc * CW + lane
        return jnp.min(jnp.where(acc <= m, j, K), axis=1)

    idx_ref[0, :] = nearest_idx(z0)
    idx_ref[1, :] = nearest_idx(z1)


def _dec_kernel(h_ref, q0_ref, q1_ref, d1_ref, c1_ref,
                d2_ref, c2_ref, d3_ref, c3_ref,
                zq_ref, qi_ref, xp_ref):
    h = h_ref[...]
    qi = jnp.concatenate([q0_ref[:, :EMB], q1_ref[:, :EMB]], axis=1)
    qi_ref[...] = qi
    zq = h + (qi - h)
    zq_ref[...] = zq
    g1 = _lrelu(_dot(zq, d1_ref[...]) + c1_ref[...])
    g2 = _lrelu(_dot(g1, d2_ref[...]) + c2_ref[...])
    xp_ref[...] = jax.nn.sigmoid(_dot(g2, d3_ref[...]) + c3_ref[...])


def _make_sc_gather(n_idx, n_rows, d):
    info = plsc.get_sparse_core_info()
    nc, ns = info.num_cores, info.num_subcores
    nw = nc * ns
    b_per_w = n_idx // nw
    slab = n_rows // ns
    chunk = 128
    n_chunks = b_per_w // chunk
    mesh = plsc.VectorSubcoreMesh(core_axis_name="c", subcore_axis_name="s")

    @functools.partial(
        pl.kernel, mesh=mesh,
        out_type=jax.ShapeDtypeStruct((n_idx, d), jnp.float32),
        scratch_types=[
            pltpu.VMEM((b_per_w,), jnp.int32),
            pltpu.VMEM((b_per_w, d), jnp.float32),
            pltpu.VMEM_SHARED((n_rows, d), jnp.float32),
            pltpu.SemaphoreType.DMA,
        ],
    )
    def sc_gather(table_hbm, idx_hbm, out_hbm, idx_v, rows_v, table_sp, sem):
        cid = lax.axis_index("c")
        sid = lax.axis_index("s")
        wid = sid * nc + cid
        pltpu.sync_copy(table_hbm.at[pl.ds(sid * slab, slab)],
                        table_sp.at[pl.ds(sid * slab, slab)])
        base = wid * b_per_w
        pltpu.sync_copy(idx_hbm.at[pl.ds(base, b_per_w)], idx_v)
        plsc.subcore_barrier()
        copies = [
            pltpu.async_copy(
                table_sp.at[idx_v.at[pl.ds(c * chunk, chunk)]],
                rows_v.at[pl.ds(c * chunk, chunk)], sem)
            for c in range(n_chunks)
        ]
        for cp in copies:
            cp.wait()
        pltpu.sync_copy(rows_v, out_hbm.at[pl.ds(base, b_per_w)])

    return sc_gather


@jax.jit
def kernel(x, W1, b1, W2, b2, W3, b3, D1, c1, D2, c2, D3, c3, emb_w):
    B = x.shape[0]
    F = x.shape[1]
    f32 = jnp.float32
    emb_wT = jnp.pad(emb_w.T, ((0, 0), (0, 128 - EMB)))
    w2 = jnp.sum(emb_w * emb_w, axis=0)[None, :]
    W3p = jnp.concatenate([W3[:, 0::2], W3[:, 1::2]], axis=1)
    b3p = jnp.concatenate([b3[0::2], b3[1::2]])[None, :]
    D1p = jnp.concatenate([D1[0::2, :], D1[1::2, :]], axis=0)

    nb = B // BB
    row_spec = lambda w: pl.BlockSpec((BB, w), lambda i: (i, 0))
    full = lambda a: pl.BlockSpec(a.shape, lambda i: (0,) * a.ndim)
    idx_spec = pl.BlockSpec((2, BB), lambda i: (0, i))

    h, idx2 = pl.pallas_call(
        _enc_kernel,
        grid=(nb,),
        in_specs=[
            row_spec(F),
            full(W1), full(b1[None, :]), full(W2), full(b2[None, :]),
            full(W3p), full(b3p), full(emb_w), full(w2),
        ],
        out_specs=[row_spec(H), idx_spec],
        out_shape=[
            jax.ShapeDtypeStruct((B, H), f32),
            jax.ShapeDtypeStruct((2, B), jnp.int32),
        ],
    )(x, W1, b1[None, :], W2, b2[None, :], W3p, b3p, emb_w, w2)

    idx_all = idx2.reshape(2 * B)

    q_all = _make_sc_gather(2 * B, K, 128)(emb_wT, idx_all)

    nbc = B // BBC
    rowc = lambda w: pl.BlockSpec((BBC, w), lambda i: (i, 0))
    q0_spec = pl.BlockSpec((BBC, 128), lambda i: (i, 0))
    q1_spec = pl.BlockSpec((BBC, 128), lambda i: (i + nbc, 0))
    zq, qi, xp = pl.pallas_call(
        _dec_kernel,
        grid=(nbc,),
        in_specs=[
            rowc(H), q0_spec, q1_spec,
            full(D1p), full(c1[None, :]),
            full(D2), full(c2[None, :]), full(D3), full(c3[None, :]),
        ],
        out_specs=[rowc(H), rowc(H), rowc(F)],
        out_shape=[
            jax.ShapeDtypeStruct((B, H), f32),
            jax.ShapeDtypeStruct((B, H), f32),
            jax.ShapeDtypeStruct((B, F), f32),
        ],
    )(h, q_all, q_all, D1p, c1[None, :], D2, c2[None, :], D3, c3[None, :])

    idx = idx2.reshape(2, B).T
    tomix = lambda a: a.reshape(B, 2, EMB).transpose(0, 2, 1)
    z_e = tomix(h)
    z_q = tomix(zq)
    emb = tomix(qi)
    return idx, z_e, z_q, emb, xp


# speedup vs baseline: 1.0640x; 1.0640x over previous
import functools

import jax
import jax.numpy as jnp
from jax import lax
from jax.experimental import pallas as pl
from jax.experimental.pallas import tpu as pltpu
from jax.experimental.pallas import tpu_sc as plsc

BB = 256
BBC = 1024
K = 8192
EMB = 32
H = 64


def _lrelu(v):
    return jnp.where(v > 0, v, 0.01 * v)


def _dot(a, b):
    return jnp.dot(a, b, preferred_element_type=jnp.float32)


CW = 128
NCH = K // CW


def _enc_kernel(x_ref, w1_ref, b1_ref, w2_ref, b2_ref, w3_ref, b3_ref,
                wemb_ref, wsq_ref, h_ref, idx_ref):
    x = x_ref[...]
    h1 = _lrelu(_dot(x, w1_ref[...]) + b1_ref[...])
    h2 = _lrelu(_dot(h1, w2_ref[...]) + b2_ref[...])
    h3 = _lrelu(_dot(h2, w3_ref[...]) + b3_ref[...])
    h_ref[...] = h3
    z0 = h3[:, :EMB]
    z1 = h3[:, EMB:]

    bb = h3.shape[0]
    lane = lax.broadcasted_iota(jnp.int32, (bb, CW), 1)

    def nearest_idx(z):
        zn = -2.0 * z
        acc = jnp.full((bb, CW), jnp.inf, jnp.float32)
        iacc = jnp.zeros((bb, CW), jnp.int32)
        for c in range(NCH):
            sl = slice(c * CW, (c + 1) * CW)
            d = _dot(zn, wemb_ref[:, sl]) + wsq_ref[:, sl]
            mask = d < acc
            acc = jnp.minimum(acc, d)
            iacc = jnp.where(mask, c, iacc)
        m = jnp.min(acc, axis=1, keepdims=True)
        j = iacc * CW + lane
        return jnp.min(jnp.where(acc <= m, j, K), axis=1)

    idx_ref[0, :] = nearest_idx(z0)
    idx_ref[1, :] = nearest_idx(z1)


def _dec_kernel(h_ref, q0_ref, q1_ref, d1_ref, c1_ref,
                d2_ref, c2_ref, d3_ref, c3_ref,
                zq_ref, qi_ref, xp_ref):
    h = h_ref[...]
    qi = jnp.concatenate([q0_ref[:, :EMB], q1_ref[:, :EMB]], axis=1)
    qi_ref[...] = qi
    zq = h + (qi - h)
    zq_ref[...] = zq
    g1 = _lrelu(_dot(zq, d1_ref[...]) + c1_ref[...])
    g2 = _lrelu(_dot(g1, d2_ref[...]) + c2_ref[...])
    xp_ref[...] = jax.nn.sigmoid(_dot(g2, d3_ref[...]) + c3_ref[...])


def _make_sc_gather(n_idx, n_rows, d):
    info = plsc.get_sparse_core_info()
    nc, ns = info.num_cores, info.num_subcores
    nw = nc * ns
    b_per_w = n_idx // nw
    slab = n_rows // ns
    chunk = 128
    n_chunks = b_per_w // chunk
    mesh = plsc.VectorSubcoreMesh(core_axis_name="c", subcore_axis_name="s")

    @functools.partial(
        pl.kernel, mesh=mesh,
        out_type=jax.ShapeDtypeStruct((n_idx, d), jnp.float32),
        scratch_types=[
            pltpu.VMEM((b_per_w,), jnp.int32),
            pltpu.VMEM((b_per_w, d), jnp.float32),
            pltpu.VMEM_SHARED((n_rows, d), jnp.float32),
            pltpu.SemaphoreType.DMA,
        ],
    )
    def sc_gather(table_hbm, idx_hbm, out_hbm, idx_v, rows_v, table_sp, sem):
        cid = lax.axis_index("c")
        sid = lax.axis_index("s")
        wid = sid * nc + cid
        pltpu.sync_copy(table_hbm.at[pl.ds(sid * slab, slab)],
                        table_sp.at[pl.ds(sid * slab, slab)])
        base = wid * b_per_w
        pltpu.sync_copy(idx_hbm.at[pl.ds(base, b_per_w)], idx_v)
        plsc.subcore_barrier()
        copies = [
            pltpu.async_copy(
                table_sp.at[idx_v.at[pl.ds(c * chunk, chunk)]],
                rows_v.at[pl.ds(c * chunk, chunk)], sem)
            for c in range(n_chunks)
        ]
        for cp in copies:
            cp.wait()
        pltpu.sync_copy(rows_v, out_hbm.at[pl.ds(base, b_per_w)])

    return sc_gather


@jax.jit
def kernel(x, W1, b1, W2, b2, W3, b3, D1, c1, D2, c2, D3, c3, emb_w):
    B = x.shape[0]
    F = x.shape[1]
    f32 = jnp.float32
    emb_wT = jnp.pad(emb_w.T, ((0, 0), (0, 128 - EMB)))
    w2 = jnp.sum(emb_w * emb_w, axis=0)[None, :]
    W3p = jnp.concatenate([W3[:, 0::2], W3[:, 1::2]], axis=1)
    b3p = jnp.concatenate([b3[0::2], b3[1::2]])[None, :]
    D1p = jnp.concatenate([D1[0::2, :], D1[1::2, :]], axis=0)

    nb = B // BB
    row_spec = lambda w: pl.BlockSpec((BB, w), lambda i: (i, 0))
    full = lambda a: pl.BlockSpec(a.shape, lambda i: (0,) * a.ndim)
    idx_spec = pl.BlockSpec((2, BB), lambda i: (0, i))

    h, idx2 = pl.pallas_call(
        _enc_kernel,
        grid=(nb,),
        in_specs=[
            row_spec(F),
            full(W1), full(b1[None, :]), full(W2), full(b2[None, :]),
            full(W3p), full(b3p), full(emb_w), full(w2),
        ],
        out_specs=[row_spec(H), idx_spec],
        out_shape=[
            jax.ShapeDtypeStruct((B, H), f32),
            jax.ShapeDtypeStruct((2, B), jnp.int32),
        ],
    )(x, W1, b1[None, :], W2, b2[None, :], W3p, b3p, emb_w, w2)

    idx_all = idx2.reshape(2 * B)

    q_all = _make_sc_gather(2 * B, K, 128)(emb_wT, idx_all)

    nbc = B // BBC
    rowc = lambda w: pl.BlockSpec((BBC, w), lambda i: (i, 0))
    q0_spec = pl.BlockSpec((BBC, 128), lambda i: (i, 0))
    q1_spec = pl.BlockSpec((BBC, 128), lambda i: (i + nbc, 0))
    zq, qi, xp = pl.pallas_call(
        _dec_kernel,
        grid=(nbc,),
        in_specs=[
            rowc(H), q0_spec, q1_spec,
            full(D1p), full(c1[None, :]),
            full(D2), full(c2[None, :]), full(D3), full(c3[None, :]),
        ],
        out_specs=[rowc(H), rowc(H), rowc(F)],
        out_shape=[
            jax.ShapeDtypeStruct((B, H), f32),
            jax.ShapeDtypeStruct((B, H), f32),
            jax.ShapeDtypeStruct((B, F), f32),
        ],
    )(h, q_all, q_all, D1p, c1[None, :], D2, c2[None, :], D3, c3[None, :])

    idx = idx2.reshape(2, B).T
    tomix = lambda a: a.reshape(B, 2, EMB).transpose(0, 2, 1)
    z_e = tomix(h)
    z_q = tomix(zq)
    emb = tomix(qi)
    return idx, z_e, z_q, emb, xp


# speedup vs baseline: 1.0843x; 1.0191x over previous
import functools

import jax
import jax.numpy as jnp
from jax import lax
from jax.experimental import pallas as pl
from jax.experimental.pallas import tpu as pltpu
from jax.experimental.pallas import tpu_sc as plsc

BB = 256
BBC = 1024
K = 8192
EMB = 32
H = 64


def _lrelu(v):
    return jnp.where(v > 0, v, 0.01 * v)


def _dot(a, b):
    return jnp.dot(a, b, preferred_element_type=jnp.float32)


CW = 128
NCH = K // CW


def _enc_kernel(x_ref, w1_ref, b1_ref, w2_ref, b2_ref, w3_ref, b3_ref,
                wemb_ref, wsq_ref, h_ref, idx_ref):
    x = x_ref[...]
    h1 = _lrelu(_dot(x, w1_ref[...]) + b1_ref[...])
    h2 = _lrelu(_dot(h1, w2_ref[...]) + b2_ref[...])
    h3 = _lrelu(_dot(h2, w3_ref[...]) + b3_ref[...])
    h_ref[...] = h3
    z0 = h3[:, :EMB]
    z1 = h3[:, EMB:]

    def nearest_idx(z):
        rb = z.shape[0]
        lane = lax.broadcasted_iota(jnp.int32, (rb, CW), 1)
        zn = -2.0 * z
        acc = jnp.full((rb, CW), jnp.inf, jnp.float32)
        iacc = jnp.zeros((rb, CW), jnp.int32)
        for c in range(NCH):
            sl = slice(c * CW, (c + 1) * CW)
            d = _dot(zn, wemb_ref[:, sl]) + wsq_ref[:, sl]
            mask = d < acc
            acc = jnp.minimum(acc, d)
            iacc = jnp.where(mask, c, iacc)
        m = jnp.min(acc, axis=1, keepdims=True)
        j = iacc * CW + lane
        return jnp.min(jnp.where(acc <= m, j, K), axis=1)

    RH = 128
    for s, z in ((0, z0), (1, z1)):
        for r in range(h3.shape[0] // RH):
            idx_ref[s, r * RH:(r + 1) * RH] = nearest_idx(
                z[r * RH:(r + 1) * RH])


def _dec_kernel(h_ref, q0_ref, q1_ref, d1_ref, c1_ref,
                d2_ref, c2_ref, d3_ref, c3_ref,
                zq_ref, qi_ref, xp_ref):
    h = h_ref[...]
    qi = jnp.concatenate([q0_ref[:, :EMB], q1_ref[:, :EMB]], axis=1)
    qi_ref[...] = qi
    zq = h + (qi - h)
    zq_ref[...] = zq
    g1 = _lrelu(_dot(zq, d1_ref[...]) + c1_ref[...])
    g2 = _lrelu(_dot(g1, d2_ref[...]) + c2_ref[...])
    xp_ref[...] = jax.nn.sigmoid(_dot(g2, d3_ref[...]) + c3_ref[...])


def _make_sc_gather(n_idx, n_rows, d):
    info = plsc.get_sparse_core_info()
    nc, ns = info.num_cores, info.num_subcores
    nw = nc * ns
    b_per_w = n_idx // nw
    slab = n_rows // ns
    chunk = 128
    n_chunks = b_per_w // chunk
    mesh = plsc.VectorSubcoreMesh(core_axis_name="c", subcore_axis_name="s")

    @functools.partial(
        pl.kernel, mesh=mesh,
        out_type=jax.ShapeDtypeStruct((n_idx, d), jnp.float32),
        scratch_types=[
            pltpu.VMEM((b_per_w,), jnp.int32),
            pltpu.VMEM((b_per_w, d), jnp.float32),
            pltpu.VMEM_SHARED((n_rows, d), jnp.float32),
            pltpu.SemaphoreType.DMA,
        ],
    )
    def sc_gather(table_hbm, idx_hbm, out_hbm, idx_v, rows_v, table_sp, sem):
        cid = lax.axis_index("c")
        sid = lax.axis_index("s")
        wid = sid * nc + cid
        pltpu.sync_copy(table_hbm.at[pl.ds(sid * slab, slab)],
                        table_sp.at[pl.ds(sid * slab, slab)])
        base = wid * b_per_w
        pltpu.sync_copy(idx_hbm.at[pl.ds(base, b_per_w)], idx_v)
        plsc.subcore_barrier()
        copies = [
            pltpu.async_copy(
                table_sp.at[idx_v.at[pl.ds(c * chunk, chunk)]],
                rows_v.at[pl.ds(c * chunk, chunk)], sem)
            for c in range(n_chunks)
        ]
        for cp in copies:
            cp.wait()
        pltpu.sync_copy(rows_v, out_hbm.at[pl.ds(base, b_per_w)])

    return sc_gather


@jax.jit
def kernel(x, W1, b1, W2, b2, W3, b3, D1, c1, D2, c2, D3, c3, emb_w):
    B = x.shape[0]
    F = x.shape[1]
    f32 = jnp.float32
    emb_wT = jnp.pad(emb_w.T, ((0, 0), (0, 128 - EMB)))
    w2 = jnp.sum(emb_w * emb_w, axis=0)[None, :]
    W3p = jnp.concatenate([W3[:, 0::2], W3[:, 1::2]], axis=1)
    b3p = jnp.concatenate([b3[0::2], b3[1::2]])[None, :]
    D1p = jnp.concatenate([D1[0::2, :], D1[1::2, :]], axis=0)

    nb = B // BB
    row_spec = lambda w: pl.BlockSpec((BB, w), lambda i: (i, 0))
    full = lambda a: pl.BlockSpec(a.shape, lambda i: (0,) * a.ndim)
    idx_spec = pl.BlockSpec((2, BB), lambda i: (0, i))

    h, idx2 = pl.pallas_call(
        _enc_kernel,
        grid=(nb,),
        in_specs=[
            row_spec(F),
            full(W1), full(b1[None, :]), full(W2), full(b2[None, :]),
            full(W3p), full(b3p), full(emb_w), full(w2),
        ],
        out_specs=[row_spec(H), idx_spec],
        out_shape=[
            jax.ShapeDtypeStruct((B, H), f32),
            jax.ShapeDtypeStruct((2, B), jnp.int32),
        ],
    )(x, W1, b1[None, :], W2, b2[None, :], W3p, b3p, emb_w, w2)

    idx_all = idx2.reshape(2 * B)

    q_all = _make_sc_gather(2 * B, K, 128)(emb_wT, idx_all)

    nbc = B // BBC
    rowc = lambda w: pl.BlockSpec((BBC, w), lambda i: (i, 0))
    q0_spec = pl.BlockSpec((BBC, 128), lambda i: (i, 0))
    q1_spec = pl.BlockSpec((BBC, 128), lambda i: (i + nbc, 0))
    zq, qi, xp = pl.pallas_call(
        _dec_kernel,
        grid=(nbc,),
        in_specs=[
            rowc(H), q0_spec, q1_spec,
            full(D1p), full(c1[None, :]),
            full(D2), full(c2[None, :]), full(D3), full(c3[None, :]),
        ],
        out_specs=[rowc(H), rowc(H), rowc(F)],
        out_shape=[
            jax.ShapeDtypeStruct((B, H), f32),
            jax.ShapeDtypeStruct((B, H), f32),
            jax.ShapeDtypeStruct((B, F), f32),
        ],
    )(h, q_all, q_all, D1p, c1[None, :], D2, c2[None, :], D3, c3[None, :])

    idx = idx2.reshape(2, B).T
    tomix = lambda a: a.reshape(B, 2, EMB).transpose(0, 2, 1)
    z_e = tomix(h)
    z_q = tomix(zq)
    emb = tomix(qi)
    return idx, z_e, z_q, emb, xp
